# baseline (speedup 1.0000x reference)
import functools

import jax
import jax.numpy as jnp
from jax import lax
from jax.experimental import pallas as pl
from jax.experimental.pallas import tpu as pltpu

_RES = 25
_DS = 16
_FPS = [1024, 256, 64]

_CP = pltpu.CompilerParams(
    dimension_semantics=("parallel",),
    vmem_limit_bytes=56 * 1024 * 1024,
)


def _pad_rows(x, m):
    pad = (-x.shape[0]) % m
    if pad:
        x = jnp.concatenate([x, jnp.zeros((pad,) + x.shape[1:], x.dtype)], 0)
    return x


def _row_mlp_kernel(x_ref, *refs, nlayers, final, ln):
    h = x_ref[...]
    i0 = 0
    if ln:
        g = refs[0][...]
        b = refs[1][...]
        i0 = 2
        m = jnp.mean(h, -1, keepdims=True)
        v = jnp.mean((h - m) ** 2, -1, keepdims=True)
        h = (h - m) * lax.rsqrt(v + 1e-5) * g + b
    out_ref = refs[-1]
    for l in range(nlayers):
        W = refs[i0 + 2 * l][...]
        bb = refs[i0 + 2 * l + 1][...]
        h = jnp.dot(h, W, preferred_element_type=jnp.float32) + bb
        if l < nlayers - 1 or final == "relu":
            h = jnp.maximum(h, 0.0)
    if final == "logsoftmax":
        mx = jnp.max(h, -1, keepdims=True)
        h = h - mx
        h = h - jnp.log(jnp.sum(jnp.exp(h), -1, keepdims=True))
    out_ref[...] = h


def _row_mlp(x, layers, final="relu", ln=None, blk=512):
    M, Cin = x.shape
    bm = min(blk, M)
    xp = _pad_rows(x, bm)
    Mp = xp.shape[0]
    ins = [xp]
    in_specs = [pl.BlockSpec((bm, Cin), lambda i: (i, 0))]
    if ln is not None:
        g, b = ln
        ins += [g.reshape(1, -1), b.reshape(1, -1)]
        in_specs += [pl.BlockSpec((1, Cin), lambda i: (0, 0))] * 2
    for W, bb in layers:
        ins += [W, bb.reshape(1, -1)]
        in_specs += [
            pl.BlockSpec(W.shape, lambda i: (0, 0)),
            pl.BlockSpec((1, W.shape[1]), lambda i: (0, 0)),
        ]
    Co = layers[-1][0].shape[1]
    out = pl.pallas_call(
        functools.partial(
            _row_mlp_kernel, nlayers=len(layers), final=final, ln=ln is not None
        ),
        out_shape=jax.ShapeDtypeStruct((Mp, Co), jnp.float32),
        grid=(Mp // bm,),
        in_specs=in_specs,
        out_specs=pl.BlockSpec((bm, Co), lambda i: (i, 0)),
        compiler_params=_CP,
        name="row_mlp",
    )(*ins)
    return out[:M]


def _sa_kernel(x_ref, *refs, nlayers):
    out_ref = refs[-1]
    sblk, ns, Cin = x_ref.shape
    h = x_ref[...].reshape(sblk * ns, Cin)
    for l in range(nlayers):
        W = refs[2 * l][...]
        bb = refs[2 * l + 1][...]
        h = jnp.maximum(jnp.dot(h, W, preferred_element_type=jnp.float32) + bb, 0.0)
    Co = h.shape[1]
    out_ref[...] = jnp.max(h.reshape(sblk, ns, Co), axis=1)


def _sa_mlp_max(gp, layers):
    BS, ns, Cin = gp.shape
    sblk = max(8, 512 // ns)
    sblk = min(sblk, BS)
    gp = _pad_rows(gp, sblk)
    BSp = gp.shape[0]
    ins = [gp]
    in_specs = [pl.BlockSpec((sblk, ns, Cin), lambda i: (i, 0, 0))]
    for W, bb in layers:
        ins += [W, bb.reshape(1, -1)]
        in_specs += [
            pl.BlockSpec(W.shape, lambda i: (0, 0)),
            pl.BlockSpec((1, W.shape[1]), lambda i: (0, 0)),
        ]
    Co = layers[-1][0].shape[1]
    out = pl.pallas_call(
        functools.partial(_sa_kernel, nlayers=len(layers)),
        out_shape=jax.ShapeDtypeStruct((BSp, Co), jnp.float32),
        grid=(BSp // sblk,),
        in_specs=in_specs,
        out_specs=pl.BlockSpec((sblk, Co), lambda i: (i, 0)),
        compiler_params=_CP,
        name="sa_mlp_max",
    )(*ins)
    return out[:BS]


def _scan_kernel(
    xz_ref, cw_ref, cb_ref, xp1_ref, xp2_ref, dtw_ref, dtb_ref, At_ref, Dp_ref,
    out_ref, dt_s, dtx_s, bt_s, ct_s, ys_s, h_s, *, L, di, dtr
):
    xz = xz_ref[0]
    xi = xz[:, :di]
    z = xz[:, di:]
    cw = cw_ref[...]
    acc = xi * cw[3:4, :]
    for j in range(3):
        sh = 3 - j
        shifted = jnp.concatenate(
            [jnp.zeros((sh, di), jnp.float32), xi[: L - sh, :]], axis=0
        )
        acc += shifted * cw[j : j + 1, :]
    xc = acc + cb_ref[...]
    xc = xc * (1.0 / (1.0 + jnp.exp(-xc)))
    dtp = jnp.dot(xc, xp1_ref[...], preferred_element_type=jnp.float32)
    bc = jnp.dot(xc, xp2_ref[...], preferred_element_type=jnp.float32)
    dtv = jnp.dot(dtp, dtw_ref[...], preferred_element_type=jnp.float32) + dtb_ref[...]
    dtv = jnp.maximum(dtv, 0.0) + jnp.log1p(jnp.exp(-jnp.abs(dtv)))
    dt_s[...] = dtv
    dtx_s[...] = dtv * xc
    bt_s[...] = bc[:, :_DS]
    ct_s[...] = bc[:, _DS:]
    h_s[...] = jnp.zeros((_DS, di), jnp.float32)
    At = At_ref[...]

    def step(t, carry):
        dt_t = dt_s[pl.ds(t, 1), :]
        dA = jnp.exp(dt_t * At)
        b_t = bt_s[pl.ds(t, 1), :]
        dtx_t = dtx_s[pl.ds(t, 1), :]
        upd = lax.dot_general(
            b_t, dtx_t, (((0,), (0,)), ((), ())),
            preferred_element_type=jnp.float32,
        )
        h = h_s[...] * dA + upd
        h_s[...] = h
        c_t = ct_s[pl.ds(t, 1), :]
        ys_s[pl.ds(t, 1), :] = lax.dot_general(
            c_t, h, (((1,), (0,)), ((), ())),
            preferred_element_type=jnp.float32,
        )
        return carry

    lax.fori_loop(0, L, step, 0)
    y = ys_s[...] + xc * Dp_ref[...]
    out_ref[0] = y * (z * (1.0 / (1.0 + jnp.exp(-z))))


def _mamba_block(x, idx, p):
    B_, L, d = x.shape
    di = p["conv_w"].shape[1]
    dtr = p["dt_w"].shape[0]
    xg = jax.vmap(lambda pb, ib: pb[ib])(x, idx)
    xz = _row_mlp(
        xg.reshape(B_ * L, d),
        [(p["in_proj"], jnp.zeros((2 * di,), jnp.float32))],
        final="none",
        ln=(p["ln_g"], p["ln_b"]),
    ).reshape(B_, L, 2 * di)
    At = (-jnp.exp(p["A_log"])).T
    xp1 = p["x_proj"][:, :dtr]
    xp2 = p["x_proj"][:, dtr:]
    f32 = jnp.float32
    y = pl.pallas_call(
        functools.partial(_scan_kernel, L=L, di=di, dtr=dtr),
        out_shape=jax.ShapeDtypeStruct((B_, L, di), f32),
        grid=(B_,),
        in_specs=[
            pl.BlockSpec((1, L, 2 * di), lambda b: (b, 0, 0)),
            pl.BlockSpec((4, di), lambda b: (0, 0)),
            pl.BlockSpec((1, di), lambda b: (0, 0)),
            pl.BlockSpec(xp1.shape, lambda b: (0, 0)),
            pl.BlockSpec(xp2.shape, lambda b: (0, 0)),
            pl.BlockSpec(p["dt_w"].shape, lambda b: (0, 0)),
            pl.BlockSpec((1, di), lambda b: (0, 0)),
            pl.BlockSpec((_DS, di), lambda b: (0, 0)),
            pl.BlockSpec((1, di), lambda b: (0, 0)),
        ],
        out_specs=pl.BlockSpec((1, L, di), lambda b: (b, 0, 0)),
        scratch_shapes=[
            pltpu.VMEM((L, di), f32),
            pltpu.VMEM((L, di), f32),
            pltpu.VMEM((L, _DS), f32),
            pltpu.VMEM((L, _DS), f32),
            pltpu.VMEM((L, di), f32),
            pltpu.VMEM((_DS, di), f32),
        ],
        compiler_params=_CP,
        name="mamba_scan",
    )(
        xz,
        p["conv_w"],
        p["conv_b"].reshape(1, -1),
        xp1,
        xp2,
        p["dt_w"],
        p["dt_b"].reshape(1, -1),
        At,
        p["Dp"].reshape(1, -1),
    )
    out = _row_mlp(
        y.reshape(B_ * L, di),
        [(p["out_proj"], jnp.zeros((d,), jnp.float32))],
        final="none",
    )
    res = xg + out.reshape(B_, L, d)
    return jax.vmap(lambda rb, ib: jnp.zeros_like(rb).at[ib].set(rb))(res, idx)


def _square_distance(a, b):
    return jnp.sum((a[:, :, None, :] - b[:, None, :, :]) ** 2, -1)


def _index_points(points, idx):
    return jax.vmap(lambda p, i: p[i])(points, idx)


def _query_ball(radius, nsample, d, Np):
    Bsz, S, _ = d.shape
    valid = d <= radius**2
    rank = jnp.cumsum(valid.astype(jnp.int32), axis=-1)
    n_ids = jnp.broadcast_to(jnp.arange(Np, dtype=jnp.int32), d.shape)
    row = (jnp.arange(Bsz * S, dtype=jnp.int32) * nsample).reshape(Bsz, S, 1)
    big = Bsz * S * nsample
    flatpos = jnp.where(
        valid & (rank <= nsample), row + rank - 1, big
    ).reshape(-1)
    out = jnp.full((big + 1,), Np, jnp.int32)
    out = out.at[flatpos].set(n_ids.reshape(-1), mode="drop")[:big]
    out = out.reshape(Bsz, S, nsample)
    first = out[:, :, :1]
    return jnp.where(out == Np, first, out)


def _sa_msg(xyz, points, fps_idx, radii, nsamples, mlps):
    new_xyz = _index_points(xyz, fps_idx)
    Bsz, S, _ = new_xyz.shape
    Np = xyz.shape[1]
    d = _square_distance(new_xyz, xyz)
    outs = []
    for r, ns, mlp in zip(radii, nsamples, mlps):
        gidx = _query_ball(r, ns, d, Np)
        gxyz = _index_points(xyz, gidx) - new_xyz[:, :, None, :]
        gp = jnp.concatenate([_index_points(points, gidx), gxyz], -1)
        Cin = gp.shape[-1]
        o = _sa_mlp_max(gp.reshape(Bsz * S, ns, Cin), mlp)
        outs.append(o.reshape(Bsz, S, -1))
    return new_xyz, jnp.concatenate(outs, -1)


def _fp(xyz1, xyz2, points1, points2, mlp):
    Bsz, Np, _ = xyz1.shape
    S = xyz2.shape[1]
    if S == 1:
        interp = jnp.broadcast_to(points2, (Bsz, Np, points2.shape[-1]))
    else:
        d = _square_distance(xyz1, xyz2)
        negd3, idx = lax.top_k(-d, 3)
        w = 1.0 / (-negd3 + 1e-8)
        w = w / jnp.sum(w, -1, keepdims=True)
        interp = jnp.sum(_index_points(points2, idx) * w[..., None], axis=2)
    new = jnp.concatenate([points1, interp], -1)
    Cin = new.shape[-1]
    out = _row_mlp(new.reshape(Bsz * Np, Cin), mlp, final="relu")
    return out.reshape(Bsz, Np, -1)


def _sparse_cnn(pts, norm, sconv):
    Bsz = pts.shape[0]
    mn = pts.min(axis=1, keepdims=True)
    mx = pts.max(axis=1, keepdims=True)
    g = (pts - mn) / (mx - mn + 1e-8) * (_RES - 1)
    gi = jnp.clip(jnp.round(g).astype(jnp.int32), 0, _RES - 1)
    flat = gi[..., 0] * _RES * _RES + gi[..., 1] * _RES + gi[..., 2]

    def vox(f, idx):
        s = jax.ops.segment_sum(f, idx, num_segments=_RES**3)
        c = jax.ops.segment_sum(jnp.ones((idx.shape[0],), f.dtype), idx, num_segments=_RES**3)
        return s / jnp.maximum(c, 1.0)[:, None]

    grid = jax.vmap(vox)(norm, flat).reshape(Bsz, _RES, _RES, _RES, norm.shape[-1])
    feats = []
    for W, bb in sconv:
        y = lax.conv_general_dilated(
            grid, W, (1, 1, 1), "SAME", dimension_numbers=("NDHWC", "DHWIO", "NDHWC")
        )
        feats.append(jax.nn.relu(y + bb))
    gf = jnp.concatenate(feats, -1).reshape(Bsz, _RES**3, -1)
    return jax.vmap(lambda gg, ii: gg[ii])(gf, flat)


def kernel(xyz, fps_index_array, series_idx_arrays, params):
    x = jnp.concatenate([xyz[:, :4, :], xyz[:, 7:, :]], 1)
    norm = x.transpose(0, 2, 1)
    pts = norm[:, :, :3]
    Bsz, N = pts.shape[0], pts.shape[1]
    emb = _sparse_cnn(pts, norm, params["sconv"])

    l1x, l1 = _sa_msg(
        pts, emb, fps_index_array[:, 0, : _FPS[0]], (0.1, 0.2, 0.4), (32, 64, 128),
        params["sa1"],
    )
    l1 = _mamba_block(l1, series_idx_arrays[:, 0, 0, : _FPS[0]], params["m1"])
    g1 = params["alpha"] * jnp.max(l1, 1)

    l2x, l2 = _sa_msg(
        l1x, l1, fps_index_array[:, 1, : _FPS[1]], (0.2, 0.4, 0.8), (32, 64, 128),
        params["sa2"],
    )
    l2 = _mamba_block(l2, series_idx_arrays[:, 1, 0, : _FPS[1]], params["m2"])
    g2 = params["beta"] * jnp.max(l2, 1)

    l3x, l3 = _sa_msg(
        l2x, l2, fps_index_array[:, 2, : _FPS[2]], (0.4, 0.8, 1.0), (16, 32, 64),
        params["sa3"],
    )
    l3 = _mamba_block(l3, series_idx_arrays[:, 2, 0, : _FPS[2]], params["m3"])
    g3 = params["gama"] * jnp.max(l3, 1)

    feat = _row_mlp(
        jnp.concatenate([l3x, l3], -1).reshape(Bsz * _FPS[2], -1),
        params["sa4"],
        final="relu",
    ).reshape(Bsz, _FPS[2], -1)
    g4 = jnp.max(feat, 1)

    glob = jnp.concatenate([g1, g2, g3, g4], -1)[:, None, :]
    l4x = jnp.zeros((Bsz, 1, 3), jnp.float32)
    l3f = _fp(l3x, l4x, l3, glob, params["fp4"])
    l2f = _fp(l2x, l3x, l2, l3f, params["fp3"])
    l1f = _fp(l1x, l2x, l1, l2f, params["fp2"])
    l0 = _fp(pts, l1x, jnp.concatenate([pts, emb], -1), l1f, params["fp1"])

    logits = _row_mlp(
        l0.reshape(Bsz * N, -1),
        [
            (params["head_w1"], params["head_b1"]),
            (params["head_w2"], params["head_b2"]),
        ],
        final="logsoftmax",
    )
    return logits.reshape(Bsz, N, -1)


# speedup vs baseline: 5.6112x; 5.6112x over previous
import functools

import jax
import jax.numpy as jnp
from jax import lax
from jax.experimental import pallas as pl
from jax.experimental.pallas import tpu as pltpu

_RES = 25
_DS = 16
_FPS = [1024, 256, 64]

_CP = pltpu.CompilerParams(
    dimension_semantics=("parallel",),
    vmem_limit_bytes=56 * 1024 * 1024,
)


def _pad_rows(x, m):
    pad = (-x.shape[0]) % m
    if pad:
        x = jnp.concatenate([x, jnp.zeros((pad,) + x.shape[1:], x.dtype)], 0)
    return x


def _row_mlp_kernel(x_ref, *refs, nlayers, final, ln):
    h = x_ref[...]
    i0 = 0
    if ln:
        g = refs[0][...]
        b = refs[1][...]
        i0 = 2
        m = jnp.mean(h, -1, keepdims=True)
        v = jnp.mean((h - m) ** 2, -1, keepdims=True)
        h = (h - m) * lax.rsqrt(v + 1e-5) * g + b
    out_ref = refs[-1]
    for l in range(nlayers):
        W = refs[i0 + 2 * l][...]
        bb = refs[i0 + 2 * l + 1][...]
        h = jnp.dot(h, W, preferred_element_type=jnp.float32) + bb
        if l < nlayers - 1 or final == "relu":
            h = jnp.maximum(h, 0.0)
    if final == "logsoftmax":
        mx = jnp.max(h, -1, keepdims=True)
        h = h - mx
        h = h - jnp.log(jnp.sum(jnp.exp(h), -1, keepdims=True))
    out_ref[...] = h


def _row_mlp(x, layers, final="relu", ln=None, blk=512):
    M, Cin = x.shape
    bm = min(blk, M)
    xp = _pad_rows(x, bm)
    Mp = xp.shape[0]
    ins = [xp]
    in_specs = [pl.BlockSpec((bm, Cin), lambda i: (i, 0))]
    if ln is not None:
        g, b = ln
        ins += [g.reshape(1, -1), b.reshape(1, -1)]
        in_specs += [pl.BlockSpec((1, Cin), lambda i: (0, 0))] * 2
    for W, bb in layers:
        ins += [W, bb.reshape(1, -1)]
        in_specs += [
            pl.BlockSpec(W.shape, lambda i: (0, 0)),
            pl.BlockSpec((1, W.shape[1]), lambda i: (0, 0)),
        ]
    Co = layers[-1][0].shape[1]
    out = pl.pallas_call(
        functools.partial(
            _row_mlp_kernel, nlayers=len(layers), final=final, ln=ln is not None
        ),
        out_shape=jax.ShapeDtypeStruct((Mp, Co), jnp.float32),
        grid=(Mp // bm,),
        in_specs=in_specs,
        out_specs=pl.BlockSpec((bm, Co), lambda i: (i, 0)),
        compiler_params=_CP,
        name="row_mlp",
    )(*ins)
    return out[:M]


def _sa_kernel(x_ref, *refs, nlayers):
    out_ref = refs[-1]
    sblk, ns, Cin = x_ref.shape
    h = x_ref[...].reshape(sblk * ns, Cin)
    for l in range(nlayers):
        W = refs[2 * l][...]
        bb = refs[2 * l + 1][...]
        h = jnp.maximum(jnp.dot(h, W, preferred_element_type=jnp.float32) + bb, 0.0)
    Co = h.shape[1]
    out_ref[...] = jnp.max(h.reshape(sblk, ns, Co), axis=1)


def _sa_mlp_max(gp, layers):
    BS, ns, Cin = gp.shape
    sblk = max(8, 512 // ns)
    sblk = min(sblk, BS)
    gp = _pad_rows(gp, sblk)
    BSp = gp.shape[0]
    ins = [gp]
    in_specs = [pl.BlockSpec((sblk, ns, Cin), lambda i: (i, 0, 0))]
    for W, bb in layers:
        ins += [W, bb.reshape(1, -1)]
        in_specs += [
            pl.BlockSpec(W.shape, lambda i: (0, 0)),
            pl.BlockSpec((1, W.shape[1]), lambda i: (0, 0)),
        ]
    Co = layers[-1][0].shape[1]
    out = pl.pallas_call(
        functools.partial(_sa_kernel, nlayers=len(layers)),
        out_shape=jax.ShapeDtypeStruct((BSp, Co), jnp.float32),
        grid=(BSp // sblk,),
        in_specs=in_specs,
        out_specs=pl.BlockSpec((sblk, Co), lambda i: (i, 0)),
        compiler_params=_CP,
        name="sa_mlp_max",
    )(*ins)
    return out[:BS]


def _scan_kernel(
    xz_ref, cw_ref, cb_ref, xp1_ref, xp2_ref, dtw_ref, dtb_ref, At_ref, Dp_ref,
    out_ref, dt_s, dtx_s, bt_s, ct_s, ys_s, h_s, *, L, di, dtr
):
    xz = xz_ref[0]
    xi = xz[:, :di]
    z = xz[:, di:]
    cw = cw_ref[...]
    acc = xi * cw[3:4, :]
    for j in range(3):
        sh = 3 - j
        shifted = jnp.concatenate(
            [jnp.zeros((sh, di), jnp.float32), xi[: L - sh, :]], axis=0
        )
        acc += shifted * cw[j : j + 1, :]
    xc = acc + cb_ref[...]
    xc = xc * (1.0 / (1.0 + jnp.exp(-xc)))
    dtp = jnp.dot(xc, xp1_ref[...], preferred_element_type=jnp.float32)
    bc = jnp.dot(xc, xp2_ref[...], preferred_element_type=jnp.float32)
    dtv = jnp.dot(dtp, dtw_ref[...], preferred_element_type=jnp.float32) + dtb_ref[...]
    dtv = jnp.maximum(dtv, 0.0) + jnp.log1p(jnp.exp(-jnp.abs(dtv)))
    dt_s[...] = dtv
    dtx_s[...] = dtv * xc
    bt_s[...] = bc[:, :_DS]
    ct_s[...] = bc[:, _DS:]
    h_s[...] = jnp.zeros((_DS, di), jnp.float32)
    At = At_ref[...]

    def step(t, carry):
        dt_t = dt_s[pl.ds(t, 1), :]
        dA = jnp.exp(dt_t * At)
        b_t = bt_s[pl.ds(t, 1), :]
        dtx_t = dtx_s[pl.ds(t, 1), :]
        upd = lax.dot_general(
            b_t, dtx_t, (((0,), (0,)), ((), ())),
            preferred_element_type=jnp.float32,
        )
        h = h_s[...] * dA + upd
        h_s[...] = h
        c_t = ct_s[pl.ds(t, 1), :]
        ys_s[pl.ds(t, 1), :] = lax.dot_general(
            c_t, h, (((1,), (0,)), ((), ())),
            preferred_element_type=jnp.float32,
        )
        return carry

    lax.fori_loop(0, L, step, 0)
    y = ys_s[...] + xc * Dp_ref[...]
    out_ref[0] = y * (z * (1.0 / (1.0 + jnp.exp(-z))))


def _mamba_block(x, idx, p):
    B_, L, d = x.shape
    di = p["conv_w"].shape[1]
    dtr = p["dt_w"].shape[0]
    xg = jax.vmap(lambda pb, ib: pb[ib])(x, idx)
    xz = _row_mlp(
        xg.reshape(B_ * L, d),
        [(p["in_proj"], jnp.zeros((2 * di,), jnp.float32))],
        final="none",
        ln=(p["ln_g"], p["ln_b"]),
    ).reshape(B_, L, 2 * di)
    At = (-jnp.exp(p["A_log"])).T
    xp1 = p["x_proj"][:, :dtr]
    xp2 = p["x_proj"][:, dtr:]
    f32 = jnp.float32
    y = pl.pallas_call(
        functools.partial(_scan_kernel, L=L, di=di, dtr=dtr),
        out_shape=jax.ShapeDtypeStruct((B_, L, di), f32),
        grid=(B_,),
        in_specs=[
            pl.BlockSpec((1, L, 2 * di), lambda b: (b, 0, 0)),
            pl.BlockSpec((4, di), lambda b: (0, 0)),
            pl.BlockSpec((1, di), lambda b: (0, 0)),
            pl.BlockSpec(xp1.shape, lambda b: (0, 0)),
            pl.BlockSpec(xp2.shape, lambda b: (0, 0)),
            pl.BlockSpec(p["dt_w"].shape, lambda b: (0, 0)),
            pl.BlockSpec((1, di), lambda b: (0, 0)),
            pl.BlockSpec((_DS, di), lambda b: (0, 0)),
            pl.BlockSpec((1, di), lambda b: (0, 0)),
        ],
        out_specs=pl.BlockSpec((1, L, di), lambda b: (b, 0, 0)),
        scratch_shapes=[
            pltpu.VMEM((L, di), f32),
            pltpu.VMEM((L, di), f32),
            pltpu.VMEM((L, _DS), f32),
            pltpu.VMEM((L, _DS), f32),
            pltpu.VMEM((L, di), f32),
            pltpu.VMEM((_DS, di), f32),
        ],
        compiler_params=_CP,
        name="mamba_scan",
    )(
        xz,
        p["conv_w"],
        p["conv_b"].reshape(1, -1),
        xp1,
        xp2,
        p["dt_w"],
        p["dt_b"].reshape(1, -1),
        At,
        p["Dp"].reshape(1, -1),
    )
    out = _row_mlp(
        y.reshape(B_ * L, di),
        [(p["out_proj"], jnp.zeros((d,), jnp.float32))],
        final="none",
    )
    res = xg + out.reshape(B_, L, d)
    return jax.vmap(lambda rb, ib: jnp.zeros_like(rb).at[ib].set(rb))(res, idx)


def _square_distance(a, b):
    return jnp.sum((a[:, :, None, :] - b[:, None, :, :]) ** 2, -1)


def _index_points(points, idx):
    return jax.vmap(lambda p, i: p[i])(points, idx)


def _query_ball(radius, nsample, d, Np):
    idx = jnp.broadcast_to(jnp.arange(Np), d.shape)
    idx = jnp.where(d > radius**2, Np, idx)
    idx = jnp.sort(idx, axis=-1)[:, :, :nsample]
    first = idx[:, :, :1]
    return jnp.where(idx == Np, first, idx)


def _sa_msg(xyz, points, fps_idx, radii, nsamples, mlps):
    new_xyz = _index_points(xyz, fps_idx)
    Bsz, S, _ = new_xyz.shape
    Np = xyz.shape[1]
    d = _square_distance(new_xyz, xyz)
    outs = []
    for r, ns, mlp in zip(radii, nsamples, mlps):
        gidx = _query_ball(r, ns, d, Np)
        gxyz = _index_points(xyz, gidx) - new_xyz[:, :, None, :]
        gp = jnp.concatenate([_index_points(points, gidx), gxyz], -1)
        Cin = gp.shape[-1]
        o = _sa_mlp_max(gp.reshape(Bsz * S, ns, Cin), mlp)
        outs.append(o.reshape(Bsz, S, -1))
    return new_xyz, jnp.concatenate(outs, -1)


def _fp(xyz1, xyz2, points1, points2, mlp):
    Bsz, Np, _ = xyz1.shape
    S = xyz2.shape[1]
    if S == 1:
        interp = jnp.broadcast_to(points2, (Bsz, Np, points2.shape[-1]))
    else:
        d = _square_distance(xyz1, xyz2)
        nd = -d
        cols = jnp.arange(d.shape[-1], dtype=jnp.int32)
        vals, idxs = [], []
        for _ in range(3):
            v = jnp.max(nd, axis=-1)
            i = jnp.argmax(nd, axis=-1)
            vals.append(v)
            idxs.append(i)
            nd = jnp.where(cols == i[..., None], -jnp.inf, nd)
        negd3 = jnp.stack(vals, -1)
        idx = jnp.stack(idxs, -1)
        w = 1.0 / (-negd3 + 1e-8)
        w = w / jnp.sum(w, -1, keepdims=True)
        interp = jnp.sum(_index_points(points2, idx) * w[..., None], axis=2)
    new = jnp.concatenate([points1, interp], -1)
    Cin = new.shape[-1]
    out = _row_mlp(new.reshape(Bsz * Np, Cin), mlp, final="relu")
    return out.reshape(Bsz, Np, -1)


def _sparse_cnn(pts, norm, sconv):
    Bsz = pts.shape[0]
    mn = pts.min(axis=1, keepdims=True)
    mx = pts.max(axis=1, keepdims=True)
    g = (pts - mn) / (mx - mn + 1e-8) * (_RES - 1)
    gi = jnp.clip(jnp.round(g).astype(jnp.int32), 0, _RES - 1)
    flat = gi[..., 0] * _RES * _RES + gi[..., 1] * _RES + gi[..., 2]

    def vox(f, idx):
        s = jax.ops.segment_sum(f, idx, num_segments=_RES**3)
        c = jax.ops.segment_sum(jnp.ones((idx.shape[0],), f.dtype), idx, num_segments=_RES**3)
        return s / jnp.maximum(c, 1.0)[:, None]

    grid = jax.vmap(vox)(norm, flat).reshape(Bsz, _RES, _RES, _RES, norm.shape[-1])
    feats = []
    for W, bb in sconv:
        y = lax.conv_general_dilated(
            grid, W, (1, 1, 1), "SAME", dimension_numbers=("NDHWC", "DHWIO", "NDHWC")
        )
        feats.append(jax.nn.relu(y + bb))
    gf = jnp.concatenate(feats, -1).reshape(Bsz, _RES**3, -1)
    return jax.vmap(lambda gg, ii: gg[ii])(gf, flat)


def kernel(xyz, fps_index_array, series_idx_arrays, params):
    x = jnp.concatenate([xyz[:, :4, :], xyz[:, 7:, :]], 1)
    norm = x.transpose(0, 2, 1)
    pts = norm[:, :, :3]
    Bsz, N = pts.shape[0], pts.shape[1]
    emb = _sparse_cnn(pts, norm, params["sconv"])

    l1x, l1 = _sa_msg(
        pts, emb, fps_index_array[:, 0, : _FPS[0]], (0.1, 0.2, 0.4), (32, 64, 128),
        params["sa1"],
    )
    l1 = _mamba_block(l1, series_idx_arrays[:, 0, 0, : _FPS[0]], params["m1"])
    g1 = params["alpha"] * jnp.max(l1, 1)

    l2x, l2 = _sa_msg(
        l1x, l1, fps_index_array[:, 1, : _FPS[1]], (0.2, 0.4, 0.8), (32, 64, 128),
        params["sa2"],
    )
    l2 = _mamba_block(l2, series_idx_arrays[:, 1, 0, : _FPS[1]], params["m2"])
    g2 = params["beta"] * jnp.max(l2, 1)

    l3x, l3 = _sa_msg(
        l2x, l2, fps_index_array[:, 2, : _FPS[2]], (0.4, 0.8, 1.0), (16, 32, 64),
        params["sa3"],
    )
    l3 = _mamba_block(l3, series_idx_arrays[:, 2, 0, : _FPS[2]], params["m3"])
    g3 = params["gama"] * jnp.max(l3, 1)

    feat = _row_mlp(
        jnp.concatenate([l3x, l3], -1).reshape(Bsz * _FPS[2], -1),
        params["sa4"],
        final="relu",
    ).reshape(Bsz, _FPS[2], -1)
    g4 = jnp.max(feat, 1)

    glob = jnp.concatenate([g1, g2, g3, g4], -1)[:, None, :]
    l4x = jnp.zeros((Bsz, 1, 3), jnp.float32)
    l3f = _fp(l3x, l4x, l3, glob, params["fp4"])
    l2f = _fp(l2x, l3x, l2, l3f, params["fp3"])
    l1f = _fp(l1x, l2x, l1, l2f, params["fp2"])
    l0 = _fp(pts, l1x, jnp.concatenate([pts, emb], -1), l1f, params["fp1"])

    logits = _row_mlp(
        l0.reshape(Bsz * N, -1),
        [
            (params["head_w1"], params["head_b1"]),
            (params["head_w2"], params["head_b2"]),
        ],
        final="logsoftmax",
    )
    return logits.reshape(Bsz, N, -1)
